# baseline (speedup 1.0000x reference)
import jax
import jax.numpy as jnp
from jax import lax
from jax.experimental import pallas as pl
from jax.experimental.pallas import tpu as pltpu
from jax.experimental.pallas import tpu_sc as plsc

_B = 4
_NPB = 262144
_N = _B * _NPB
_NW = 32
_PER_W = _N // _NW
_CHUNK = 2048
_NCHUNK = _PER_W // _CHUNK
_GROUPS = _CHUNK // 16

_VSX, _VSY, _VSZ = 0.1, 0.1, 0.2
_MNX, _MNY, _MNZ = -51.2, -51.2, -5.0
_GX, _GY, _GZ = 1024, 1024, 40


def _compute_chunk(inb, pb, cb, ib, ob, idx_base):
    iota = lax.iota(jnp.int32, 16)

    @plsc.parallel_loop(0, _GROUPS, step=1, unroll=4)
    def step(g):
        blk = g >> 3
        j = g & 7
        base = blk * 512 + j * 16
        gx = inb[pl.ds(base, 16)]
        gy = inb[pl.ds(base + 128, 16)]
        gz = inb[pl.ds(base + 256, 16)]
        gw = inb[pl.ds(base + 384, 16)]
        an = (gx != gx) | (gy != gy) | (gz != gz) | (gw != gw)
        x = jnp.where(an, 0.0, gx)
        y = jnp.where(an, 0.0, gy)
        z = jnp.where(an, 0.0, gz)
        w = jnp.where(an, 0.0, gw)
        qx = (x - _MNX) / _VSX
        qy = (y - _MNY) / _VSY
        qz = (z - _MNZ) / _VSZ
        cx = qx.astype(jnp.int32)
        cy = qy.astype(jnp.int32)
        cz = qz.astype(jnp.int32)
        cx = jnp.where(cx.astype(jnp.float32) > qx, cx - 1, cx)
        cy = jnp.where(cy.astype(jnp.float32) > qy, cy - 1, cy)
        cz = jnp.where(cz.astype(jnp.float32) > qz, cz - 1, cz)
        valid = ((cx >= 0) & (cx < _GX)
                 & (cy >= 0) & (cy < _GY)
                 & (cz >= 0) & (cz < _GZ))
        zf = jnp.zeros((16,), jnp.float32)
        zi = jnp.zeros((16,), jnp.int32)
        px = jnp.where(valid, x, zf)
        py = jnp.where(valid, y, zf)
        pz = jnp.where(valid, z, zf)
        pw = jnp.where(valid, w, zf)
        vcx = jnp.where(valid, cx, zi)
        vcy = jnp.where(valid, cy, zi)
        vcz = jnp.where(valid, cz, zi)
        vi = jnp.where(valid & (~an), iota + (idx_base + g * 16), zi)
        cenx = vcx.astype(jnp.float32) * _VSX + _MNX + _VSX / 2.0
        ceny = vcy.astype(jnp.float32) * _VSY + _MNY + _VSY / 2.0
        cenz = vcz.astype(jnp.float32) * _VSZ + _MNZ + _VSZ / 2.0
        pb[pl.ds(base, 16)] = px
        pb[pl.ds(base + 128, 16)] = py
        pb[pl.ds(base + 256, 16)] = pz
        pb[pl.ds(base + 384, 16)] = pw
        cb[pl.ds(base, 16)] = vcz
        cb[pl.ds(base + 128, 16)] = vcy
        cb[pl.ds(base + 256, 16)] = vcx
        ob[pl.ds(base, 16)] = px - cenx
        ob[pl.ds(base + 128, 16)] = py - ceny
        ob[pl.ds(base + 256, 16)] = pz - cenz
        ib[pl.ds(g * 16, 16)] = vi


def _voxel_body(pts, po, co, io, oo, *scr):
    bufs = (scr[0:5], scr[5:10])
    insem = scr[10:12]
    outsem = scr[12:14]
    wid = lax.axis_index("s") * 2 + lax.axis_index("c")
    base = wid * _PER_W
    lbase = lax.rem(base, _NPB)

    def start_in(it):
        b = it % 2
        off = (base + it * _CHUNK) * 4
        return pltpu.async_copy(
            pts.at[pl.ds(off, _CHUNK * 4)], bufs[b][0], insem[b])

    inh = [start_in(0), None]
    outh = [None, None]
    for it in range(_NCHUNK):
        b = it % 2
        if it + 1 < _NCHUNK:
            inh[(it + 1) % 2] = start_in(it + 1)
        if outh[b] is not None:
            for h in outh[b]:
                h.wait()
        inh[b].wait()
        inb, pb, cb, ib, ob = bufs[b]
        _compute_chunk(inb, pb, cb, ib, ob, lbase + it * _CHUNK)
        op = base + it * _CHUNK
        outh[b] = [
            pltpu.async_copy(pb, po.at[pl.ds(op * 4, _CHUNK * 4)], outsem[b]),
            pltpu.async_copy(cb, co.at[pl.ds(op * 4, _CHUNK * 4)], outsem[b]),
            pltpu.async_copy(ib, io.at[pl.ds(op, _CHUNK)], outsem[b]),
            pltpu.async_copy(ob, oo.at[pl.ds(op * 4, _CHUNK * 4)], outsem[b]),
        ]
    for b in (0, 1):
        for h in outh[b]:
            h.wait()


_voxelize = pl.kernel(
    _voxel_body,
    out_type=(
        jax.ShapeDtypeStruct((_N * 4,), jnp.float32),
        jax.ShapeDtypeStruct((_N * 4,), jnp.int32),
        jax.ShapeDtypeStruct((_N,), jnp.int32),
        jax.ShapeDtypeStruct((_N * 4,), jnp.float32),
    ),
    mesh=plsc.VectorSubcoreMesh(core_axis_name="c", subcore_axis_name="s",
                                num_cores=2, num_subcores=16),
    scratch_types=[
        pltpu.VMEM((_CHUNK * 4,), jnp.float32),
        pltpu.VMEM((_CHUNK * 4,), jnp.float32),
        pltpu.VMEM((_CHUNK * 4,), jnp.int32),
        pltpu.VMEM((_CHUNK,), jnp.int32),
        pltpu.VMEM((_CHUNK * 4,), jnp.float32),
        pltpu.VMEM((_CHUNK * 4,), jnp.float32),
        pltpu.VMEM((_CHUNK * 4,), jnp.float32),
        pltpu.VMEM((_CHUNK * 4,), jnp.int32),
        pltpu.VMEM((_CHUNK,), jnp.int32),
        pltpu.VMEM((_CHUNK * 4,), jnp.float32),
        pltpu.SemaphoreType.DMA,
        pltpu.SemaphoreType.DMA,
        pltpu.SemaphoreType.DMA,
        pltpu.SemaphoreType.DMA,
    ],
    compiler_params=pltpu.CompilerParams(needs_layout_passes=False),
)


def kernel(points):
    flat = points.reshape(_B, _NPB // 128, 128, 4).transpose(0, 1, 3, 2).reshape(-1)
    p, c, i, o = _voxelize(flat)
    p4 = p.reshape(_N // 128, 4, 128).transpose(0, 2, 1).reshape(_N, 4)
    c4 = c.reshape(_N // 128, 4, 128).transpose(0, 2, 1).reshape(_N, 4)
    o4 = o.reshape(_N // 128, 4, 128).transpose(0, 2, 1).reshape(_N, 4)
    return (
        p4,
        c4[:, :3].astype(jnp.int64),
        i.astype(jnp.int64),
        o4[:, :3],
    )


# speedup vs baseline: 1.0050x; 1.0050x over previous
import jax
import jax.numpy as jnp
from jax import lax
from jax.experimental import pallas as pl
from jax.experimental.pallas import tpu as pltpu
from jax.experimental.pallas import tpu_sc as plsc

_B = 4
_NPB = 262144
_N = _B * _NPB
_NW = 32
_PER_W = _N // _NW
_CHUNK = 2048
_NCHUNK = _PER_W // _CHUNK
_GROUPS = _CHUNK // 16

_VSX, _VSY, _VSZ = 0.1, 0.1, 0.2
_MNX, _MNY, _MNZ = -51.2, -51.2, -5.0
_GX, _GY, _GZ = 1024, 1024, 40


def _compute_chunk(inb, pb, cb, ib, ob, idx_base):
    iota = lax.iota(jnp.int32, 16)

    @plsc.parallel_loop(0, _GROUPS, step=1, unroll=2)
    def step(g):
        blk = g >> 3
        j = g & 7
        base = blk * 512 + j * 16
        gx = inb[pl.ds(base, 16)]
        gy = inb[pl.ds(base + 128, 16)]
        gz = inb[pl.ds(base + 256, 16)]
        gw = inb[pl.ds(base + 384, 16)]
        an = (gx != gx) | (gy != gy) | (gz != gz) | (gw != gw)
        x = jnp.where(an, 0.0, gx)
        y = jnp.where(an, 0.0, gy)
        z = jnp.where(an, 0.0, gz)
        w = jnp.where(an, 0.0, gw)
        qx = (x - _MNX) * (1.0 / _VSX)
        qy = (y - _MNY) * (1.0 / _VSY)
        qz = (z - _MNZ) * (1.0 / _VSZ)
        cx = qx.astype(jnp.int32)
        cy = qy.astype(jnp.int32)
        cz = qz.astype(jnp.int32)
        cx = jnp.where(cx.astype(jnp.float32) > qx, cx - 1, cx)
        cy = jnp.where(cy.astype(jnp.float32) > qy, cy - 1, cy)
        cz = jnp.where(cz.astype(jnp.float32) > qz, cz - 1, cz)
        valid = ((cx >= 0) & (cx < _GX)
                 & (cy >= 0) & (cy < _GY)
                 & (cz >= 0) & (cz < _GZ))
        zf = jnp.zeros((16,), jnp.float32)
        zi = jnp.zeros((16,), jnp.int32)
        px = jnp.where(valid, x, zf)
        py = jnp.where(valid, y, zf)
        pz = jnp.where(valid, z, zf)
        pw = jnp.where(valid, w, zf)
        vcx = jnp.where(valid, cx, zi)
        vcy = jnp.where(valid, cy, zi)
        vcz = jnp.where(valid, cz, zi)
        vi = jnp.where(valid & (~an), iota + (idx_base + g * 16), zi)
        cenx = vcx.astype(jnp.float32) * _VSX + _MNX + _VSX / 2.0
        ceny = vcy.astype(jnp.float32) * _VSY + _MNY + _VSY / 2.0
        cenz = vcz.astype(jnp.float32) * _VSZ + _MNZ + _VSZ / 2.0
        pb[pl.ds(base, 16)] = px
        pb[pl.ds(base + 128, 16)] = py
        pb[pl.ds(base + 256, 16)] = pz
        pb[pl.ds(base + 384, 16)] = pw
        cb[pl.ds(base, 16)] = vcz
        cb[pl.ds(base + 128, 16)] = vcy
        cb[pl.ds(base + 256, 16)] = vcx
        ob[pl.ds(base, 16)] = px - cenx
        ob[pl.ds(base + 128, 16)] = py - ceny
        ob[pl.ds(base + 256, 16)] = pz - cenz
        ib[pl.ds(g * 16, 16)] = vi


def _voxel_body(pts, po, co, io, oo, *scr):
    bufs = (scr[0:5], scr[5:10])
    insem = scr[10:12]
    outsem = scr[12:14]
    wid = lax.axis_index("s") * 2 + lax.axis_index("c")
    base = wid * _PER_W
    lbase = lax.rem(base, _NPB)

    def start_in(it):
        b = it % 2
        off = (base + it * _CHUNK) * 4
        return pltpu.async_copy(
            pts.at[pl.ds(off, _CHUNK * 4)], bufs[b][0], insem[b])

    inh = [start_in(0), None]
    outh = [None, None]
    for it in range(_NCHUNK):
        b = it % 2
        if it + 1 < _NCHUNK:
            inh[(it + 1) % 2] = start_in(it + 1)
        if outh[b] is not None:
            for h in outh[b]:
                h.wait()
        inh[b].wait()
        inb, pb, cb, ib, ob = bufs[b]
        _compute_chunk(inb, pb, cb, ib, ob, lbase + it * _CHUNK)
        op = base + it * _CHUNK
        outh[b] = [
            pltpu.async_copy(pb, po.at[pl.ds(op * 4, _CHUNK * 4)], outsem[b]),
            pltpu.async_copy(cb, co.at[pl.ds(op * 4, _CHUNK * 4)], outsem[b]),
            pltpu.async_copy(ib, io.at[pl.ds(op, _CHUNK)], outsem[b]),
            pltpu.async_copy(ob, oo.at[pl.ds(op * 4, _CHUNK * 4)], outsem[b]),
        ]
    for b in (0, 1):
        for h in outh[b]:
            h.wait()


_voxelize = pl.kernel(
    _voxel_body,
    out_type=(
        jax.ShapeDtypeStruct((_N * 4,), jnp.float32),
        jax.ShapeDtypeStruct((_N * 4,), jnp.int32),
        jax.ShapeDtypeStruct((_N,), jnp.int32),
        jax.ShapeDtypeStruct((_N * 4,), jnp.float32),
    ),
    mesh=plsc.VectorSubcoreMesh(core_axis_name="c", subcore_axis_name="s",
                                num_cores=2, num_subcores=16),
    scratch_types=[
        pltpu.VMEM((_CHUNK * 4,), jnp.float32),
        pltpu.VMEM((_CHUNK * 4,), jnp.float32),
        pltpu.VMEM((_CHUNK * 4,), jnp.int32),
        pltpu.VMEM((_CHUNK,), jnp.int32),
        pltpu.VMEM((_CHUNK * 4,), jnp.float32),
        pltpu.VMEM((_CHUNK * 4,), jnp.float32),
        pltpu.VMEM((_CHUNK * 4,), jnp.float32),
        pltpu.VMEM((_CHUNK * 4,), jnp.int32),
        pltpu.VMEM((_CHUNK,), jnp.int32),
        pltpu.VMEM((_CHUNK * 4,), jnp.float32),
        pltpu.SemaphoreType.DMA,
        pltpu.SemaphoreType.DMA,
        pltpu.SemaphoreType.DMA,
        pltpu.SemaphoreType.DMA,
    ],
    compiler_params=pltpu.CompilerParams(needs_layout_passes=False),
)


def kernel(points):
    flat = points.reshape(_B, _NPB // 128, 128, 4).transpose(0, 1, 3, 2).reshape(-1)
    p, c, i, o = _voxelize(flat)
    p4 = p.reshape(_N // 128, 4, 128).transpose(0, 2, 1).reshape(_N, 4)
    c4 = c.reshape(_N // 128, 4, 128).transpose(0, 2, 1).reshape(_N, 4)
    o4 = o.reshape(_N // 128, 4, 128).transpose(0, 2, 1).reshape(_N, 4)
    return (
        p4,
        c4[:, :3].astype(jnp.int64),
        i.astype(jnp.int64),
        o4[:, :3],
    )


# speedup vs baseline: 1.0261x; 1.0210x over previous
import jax
import jax.numpy as jnp
from jax import lax
from jax.experimental import pallas as pl
from jax.experimental.pallas import tpu as pltpu
from jax.experimental.pallas import tpu_sc as plsc

_B = 4
_NPB = 262144
_N = _B * _NPB
_NW = 32
_PER_W = _N // _NW
_CHUNK = 2048
_NCHUNK = _PER_W // _CHUNK
_GROUPS = _CHUNK // 16

_VSX, _VSY, _VSZ = 0.1, 0.1, 0.2
_MNX, _MNY, _MNZ = -51.2, -51.2, -5.0
_GX, _GY, _GZ = 1024, 1024, 40


def _compute_chunk(inb, pb, cb, ib, ob, idx_base):
    iota = lax.iota(jnp.int32, 16)

    @plsc.parallel_loop(0, _GROUPS, step=1, unroll=2)
    def step(g):
        blk = g >> 3
        j = g & 7
        base = blk * 512 + j * 16
        gx = inb[pl.ds(base, 16)]
        gy = inb[pl.ds(base + 128, 16)]
        gz = inb[pl.ds(base + 256, 16)]
        gw = inb[pl.ds(base + 384, 16)]
        an = (gx != gx) | (gy != gy) | (gz != gz) | (gw != gw)
        x = jnp.where(an, 0.0, gx)
        y = jnp.where(an, 0.0, gy)
        z = jnp.where(an, 0.0, gz)
        w = jnp.where(an, 0.0, gw)
        qx = (x - _MNX) * (1.0 / _VSX)
        qy = (y - _MNY) * (1.0 / _VSY)
        qz = (z - _MNZ) * (1.0 / _VSZ)
        cx = qx.astype(jnp.int32)
        cy = qy.astype(jnp.int32)
        cz = qz.astype(jnp.int32)
        cx = jnp.where(cx.astype(jnp.float32) > qx, cx - 1, cx)
        cy = jnp.where(cy.astype(jnp.float32) > qy, cy - 1, cy)
        cz = jnp.where(cz.astype(jnp.float32) > qz, cz - 1, cz)
        valid = ((cx >= 0) & (cx < _GX)
                 & (cy >= 0) & (cy < _GY)
                 & (cz >= 0) & (cz < _GZ))
        zf = jnp.zeros((16,), jnp.float32)
        zi = jnp.zeros((16,), jnp.int32)
        px = jnp.where(valid, x, zf)
        py = jnp.where(valid, y, zf)
        pz = jnp.where(valid, z, zf)
        pw = jnp.where(valid, w, zf)
        vcx = jnp.where(valid, cx, zi)
        vcy = jnp.where(valid, cy, zi)
        vcz = jnp.where(valid, cz, zi)
        vi = jnp.where(valid & (~an), iota + (idx_base + g * 16), zi)
        cenx = vcx.astype(jnp.float32) * _VSX + _MNX + _VSX / 2.0
        ceny = vcy.astype(jnp.float32) * _VSY + _MNY + _VSY / 2.0
        cenz = vcz.astype(jnp.float32) * _VSZ + _MNZ + _VSZ / 2.0
        pb[pl.ds(base, 16)] = px
        pb[pl.ds(base + 128, 16)] = py
        pb[pl.ds(base + 256, 16)] = pz
        pb[pl.ds(base + 384, 16)] = pw
        b3 = j * 16
        cb[blk, pl.ds(b3, 16)] = vcz
        cb[blk, pl.ds(b3 + 128, 16)] = vcy
        cb[blk, pl.ds(b3 + 256, 16)] = vcx
        ob[blk, pl.ds(b3, 16)] = px - cenx
        ob[blk, pl.ds(b3 + 128, 16)] = py - ceny
        ob[blk, pl.ds(b3 + 256, 16)] = pz - cenz
        ib[pl.ds(g * 16, 16)] = vi


def _voxel_body(pts, po, co, io, oo, *scr):
    bufs = (scr[0:5], scr[5:10])
    insem = scr[10:12]
    outsem = scr[12:14]
    wid = lax.axis_index("s") * 2 + lax.axis_index("c")
    base = wid * _PER_W
    lbase = lax.rem(base, _NPB)

    def start_in(it):
        b = it % 2
        off = (base + it * _CHUNK) * 4
        return pltpu.async_copy(
            pts.at[pl.ds(off, _CHUNK * 4)], bufs[b][0], insem[b])

    inh = [start_in(0), None]
    outh = [None, None]
    for it in range(_NCHUNK):
        b = it % 2
        if it + 1 < _NCHUNK:
            inh[(it + 1) % 2] = start_in(it + 1)
        if outh[b] is not None:
            for h in outh[b]:
                h.wait()
        inh[b].wait()
        inb, pb, cb, ib, ob = bufs[b]
        _compute_chunk(inb, pb, cb, ib, ob, lbase + it * _CHUNK)
        op = base + it * _CHUNK
        row = op // 128
        outh[b] = [
            pltpu.async_copy(pb, po.at[pl.ds(op * 4, _CHUNK * 4)], outsem[b]),
            pltpu.async_copy(
                cb, co.at[pl.ds(row, _CHUNK // 128), pl.ds(0, 384)], outsem[b]),
            pltpu.async_copy(ib, io.at[pl.ds(op, _CHUNK)], outsem[b]),
            pltpu.async_copy(
                ob, oo.at[pl.ds(row, _CHUNK // 128), pl.ds(0, 384)], outsem[b]),
        ]
    for b in (0, 1):
        for h in outh[b]:
            h.wait()


_voxelize = pl.kernel(
    _voxel_body,
    out_type=(
        jax.ShapeDtypeStruct((_N * 4,), jnp.float32),
        jax.ShapeDtypeStruct((_N // 128, 512), jnp.int32),
        jax.ShapeDtypeStruct((_N,), jnp.int32),
        jax.ShapeDtypeStruct((_N // 128, 512), jnp.float32),
    ),
    mesh=plsc.VectorSubcoreMesh(core_axis_name="c", subcore_axis_name="s",
                                num_cores=2, num_subcores=16),
    scratch_types=[
        pltpu.VMEM((_CHUNK * 4,), jnp.float32),
        pltpu.VMEM((_CHUNK * 4,), jnp.float32),
        pltpu.VMEM((_CHUNK // 128, 384), jnp.int32),
        pltpu.VMEM((_CHUNK,), jnp.int32),
        pltpu.VMEM((_CHUNK // 128, 384), jnp.float32),
        pltpu.VMEM((_CHUNK * 4,), jnp.float32),
        pltpu.VMEM((_CHUNK * 4,), jnp.float32),
        pltpu.VMEM((_CHUNK // 128, 384), jnp.int32),
        pltpu.VMEM((_CHUNK,), jnp.int32),
        pltpu.VMEM((_CHUNK // 128, 384), jnp.float32),
        pltpu.SemaphoreType.DMA,
        pltpu.SemaphoreType.DMA,
        pltpu.SemaphoreType.DMA,
        pltpu.SemaphoreType.DMA,
    ],
    compiler_params=pltpu.CompilerParams(needs_layout_passes=False, use_tc_tiling_on_sc=False),
)


def kernel(points):
    flat = points.reshape(_B, _NPB // 128, 128, 4).transpose(0, 1, 3, 2).reshape(-1)
    p, c, i, o = _voxelize(flat)
    p4 = p.reshape(_N // 128, 4, 128).transpose(0, 2, 1).reshape(_N, 4)
    c4 = c.reshape(_N // 128, 4, 128).transpose(0, 2, 1).reshape(_N, 4)
    o4 = o.reshape(_N // 128, 4, 128).transpose(0, 2, 1).reshape(_N, 4)

    return (
        p4,
        c4[:, :3].astype(jnp.int64),
        i.astype(jnp.int64),
        o4[:, :3],
    )


# speedup vs baseline: 1.0991x; 1.0712x over previous
import jax
import jax.numpy as jnp
from jax import lax
from jax.experimental import pallas as pl
from jax.experimental.pallas import tpu as pltpu
from jax.experimental.pallas import tpu_sc as plsc

_B = 4
_NPB = 262144
_N = _B * _NPB
_NW = 32
_PER_W = _N // _NW
_CHUNK = 4096
_NCHUNK = _PER_W // _CHUNK
_GROUPS = _CHUNK // 16

_VSX, _VSY, _VSZ = 0.1, 0.1, 0.2
_MNX, _MNY, _MNZ = -51.2, -51.2, -5.0
_GX, _GY, _GZ = 1024, 1024, 40


def _compute_chunk(inb, pb, cb, ib, ob, idx_base):
    iota = lax.iota(jnp.int32, 16)

    @plsc.parallel_loop(0, _GROUPS, step=1, unroll=2)
    def step(g):
        blk = g >> 3
        j = g & 7
        base = blk * 512 + j * 16
        gx = inb[pl.ds(base, 16)]
        gy = inb[pl.ds(base + 128, 16)]
        gz = inb[pl.ds(base + 256, 16)]
        gw = inb[pl.ds(base + 384, 16)]
        an = (gx != gx) | (gy != gy) | (gz != gz) | (gw != gw)
        x = jnp.where(an, 0.0, gx)
        y = jnp.where(an, 0.0, gy)
        z = jnp.where(an, 0.0, gz)
        w = jnp.where(an, 0.0, gw)
        qx = (x - _MNX) * (1.0 / _VSX)
        qy = (y - _MNY) * (1.0 / _VSY)
        qz = (z - _MNZ) * (1.0 / _VSZ)
        cx = qx.astype(jnp.int32)
        cy = qy.astype(jnp.int32)
        cz = qz.astype(jnp.int32)
        cx = jnp.where(cx.astype(jnp.float32) > qx, cx - 1, cx)
        cy = jnp.where(cy.astype(jnp.float32) > qy, cy - 1, cy)
        cz = jnp.where(cz.astype(jnp.float32) > qz, cz - 1, cz)
        valid = ((cx >= 0) & (cx < _GX)
                 & (cy >= 0) & (cy < _GY)
                 & (cz >= 0) & (cz < _GZ))
        zf = jnp.zeros((16,), jnp.float32)
        zi = jnp.zeros((16,), jnp.int32)
        px = jnp.where(valid, x, zf)
        py = jnp.where(valid, y, zf)
        pz = jnp.where(valid, z, zf)
        pw = jnp.where(valid, w, zf)
        vcx = jnp.where(valid, cx, zi)
        vcy = jnp.where(valid, cy, zi)
        vcz = jnp.where(valid, cz, zi)
        vi = jnp.where(valid & (~an), iota + (idx_base + g * 16), zi)
        cenx = vcx.astype(jnp.float32) * _VSX + _MNX + _VSX / 2.0
        ceny = vcy.astype(jnp.float32) * _VSY + _MNY + _VSY / 2.0
        cenz = vcz.astype(jnp.float32) * _VSZ + _MNZ + _VSZ / 2.0
        pb[pl.ds(base, 16)] = px
        pb[pl.ds(base + 128, 16)] = py
        pb[pl.ds(base + 256, 16)] = pz
        pb[pl.ds(base + 384, 16)] = pw
        b3 = j * 16
        cb[blk, pl.ds(b3, 16)] = vcz
        cb[blk, pl.ds(b3 + 128, 16)] = vcy
        cb[blk, pl.ds(b3 + 256, 16)] = vcx
        ob[blk, pl.ds(b3, 16)] = px - cenx
        ob[blk, pl.ds(b3 + 128, 16)] = py - ceny
        ob[blk, pl.ds(b3 + 256, 16)] = pz - cenz
        ib[pl.ds(g * 16, 16)] = vi


def _voxel_body(pts, po, co, io, oo, *scr):
    bufs = (scr[0:5], scr[5:10])
    insem = scr[10:12]
    outsem = scr[12:14]
    wid = lax.axis_index("s") * 2 + lax.axis_index("c")
    base = wid * _PER_W
    lbase = lax.rem(base, _NPB)

    def start_in(it):
        b = it % 2
        off = (base + it * _CHUNK) * 4
        return pltpu.async_copy(
            pts.at[pl.ds(off, _CHUNK * 4)], bufs[b][0], insem[b])

    inh = [start_in(0), None]
    outh = [None, None]
    for it in range(_NCHUNK):
        b = it % 2
        if it + 1 < _NCHUNK:
            inh[(it + 1) % 2] = start_in(it + 1)
        if outh[b] is not None:
            for h in outh[b]:
                h.wait()
        inh[b].wait()
        inb, pb, cb, ib, ob = bufs[b]
        _compute_chunk(inb, pb, cb, ib, ob, lbase + it * _CHUNK)
        op = base + it * _CHUNK
        row = op // 128
        outh[b] = [
            pltpu.async_copy(pb, po.at[pl.ds(op * 4, _CHUNK * 4)], outsem[b]),
            pltpu.async_copy(
                cb, co.at[pl.ds(row, _CHUNK // 128), pl.ds(0, 384)], outsem[b]),
            pltpu.async_copy(ib, io.at[pl.ds(op, _CHUNK)], outsem[b]),
            pltpu.async_copy(
                ob, oo.at[pl.ds(row, _CHUNK // 128), pl.ds(0, 384)], outsem[b]),
        ]
    for b in (0, 1):
        for h in outh[b]:
            h.wait()


_voxelize = pl.kernel(
    _voxel_body,
    out_type=(
        jax.ShapeDtypeStruct((_N * 4,), jnp.float32),
        jax.ShapeDtypeStruct((_N // 128, 512), jnp.int32),
        jax.ShapeDtypeStruct((_N,), jnp.int32),
        jax.ShapeDtypeStruct((_N // 128, 512), jnp.float32),
    ),
    mesh=plsc.VectorSubcoreMesh(core_axis_name="c", subcore_axis_name="s",
                                num_cores=2, num_subcores=16),
    scratch_types=[
        pltpu.VMEM((_CHUNK * 4,), jnp.float32),
        pltpu.VMEM((_CHUNK * 4,), jnp.float32),
        pltpu.VMEM((_CHUNK // 128, 384), jnp.int32),
        pltpu.VMEM((_CHUNK,), jnp.int32),
        pltpu.VMEM((_CHUNK // 128, 384), jnp.float32),
        pltpu.VMEM((_CHUNK * 4,), jnp.float32),
        pltpu.VMEM((_CHUNK * 4,), jnp.float32),
        pltpu.VMEM((_CHUNK // 128, 384), jnp.int32),
        pltpu.VMEM((_CHUNK,), jnp.int32),
        pltpu.VMEM((_CHUNK // 128, 384), jnp.float32),
        pltpu.SemaphoreType.DMA,
        pltpu.SemaphoreType.DMA,
        pltpu.SemaphoreType.DMA,
        pltpu.SemaphoreType.DMA,
    ],
    compiler_params=pltpu.CompilerParams(needs_layout_passes=False, use_tc_tiling_on_sc=False),
)


def kernel(points):
    flat = points.reshape(_B, _NPB // 128, 128, 4).transpose(0, 1, 3, 2).reshape(-1)
    p, c, i, o = _voxelize(flat)
    p4 = p.reshape(_N // 128, 4, 128).transpose(0, 2, 1).reshape(_N, 4)
    c4 = c.reshape(_N // 128, 4, 128).transpose(0, 2, 1).reshape(_N, 4)
    o4 = o.reshape(_N // 128, 4, 128).transpose(0, 2, 1).reshape(_N, 4)

    return (
        p4,
        c4[:, :3].astype(jnp.int64),
        i.astype(jnp.int64),
        o4[:, :3],
    )
